# baseline (speedup 1.0000x reference)
import functools

import jax
import jax.numpy as jnp
from jax import lax
from jax.experimental import pallas as pl
from jax.experimental.pallas import tpu as pltpu
from jax.experimental.pallas import tpu_sc as plsc

B, N, C = 32, 1024, 96
NUM_PROMPT = 10
NUM_HEADS = 8
HD = C // NUM_HEADS
K = 256
NKEYS = N + NUM_PROMPT
PPAD = 16
NPAD = N + PPAD
CPAD = 128

_NEG = -1e30


def _row(shape):
    return pl.BlockSpec(shape, lambda b: tuple(0 for _ in shape))


def _attn_body(x_ref, prm_ref, m1_ref, v1_ref, mp_ref, vp_ref, ln1g_ref,
               ln1b_ref, wq_ref, wk_ref, wv_ref, wproj_ref, bproj_ref,
               h1_ref, xpad_ref):
    xq = x_ref[0]
    xpad_ref[0] = jnp.concatenate(
        [xq, jnp.zeros((N, CPAD - C), jnp.float32)], axis=1)
    xp = jnp.concatenate([xq, prm_ref[...]], axis=0)
    mu = jnp.concatenate([m1_ref[0], mp_ref[...]], axis=0)
    va = jnp.concatenate([v1_ref[0], vp_ref[...]], axis=0)
    ln1 = (xp - mu) / jnp.sqrt(va + 1e-5) * ln1g_ref[...] + ln1b_ref[...]
    lnq = ln1[0:N]

    qf = jnp.dot(lnq, wq_ref[...],
                 preferred_element_type=jnp.float32) * (HD ** -0.5)
    kf = jnp.dot(ln1, wk_ref[...], preferred_element_type=jnp.float32)
    vf = jnp.dot(ln1, wv_ref[...], preferred_element_type=jnp.float32)

    colmask = lax.broadcasted_iota(jnp.int32, (1, NPAD), 1) < NKEYS

    outs = []
    for h in range(NUM_HEADS):
        qh = qf[:, h * HD:(h + 1) * HD]
        kh = kf[:, h * HD:(h + 1) * HD]
        vh = vf[:, h * HD:(h + 1) * HD]
        logits = lax.dot_general(qh, kh, (((1,), (1,)), ((), ())),
                                 preferred_element_type=jnp.float32)
        logits = jnp.where(colmask, logits, _NEG)
        mx = jnp.max(logits, axis=1, keepdims=True)
        e = jnp.exp(logits - mx)
        s = jnp.sum(e, axis=1, keepdims=True)
        probs = e / s
        oh = jnp.dot(probs, vh, preferred_element_type=jnp.float32)
        outs.append(oh.astype(jnp.bfloat16).astype(jnp.float32))
    o = jnp.concatenate(outs, axis=1)
    h1_ref[0] = xq + (jnp.dot(o, wproj_ref[...],
                              preferred_element_type=jnp.float32)
                      + bproj_ref[...])


def _attn_call(x, prm16, m1d, v1d, m1p, v1p, ln1g, ln1b, w_q, w_k, w_v,
               w_proj, bproj):
    return pl.pallas_call(
        _attn_body,
        grid=(B,),
        in_specs=[
            pl.BlockSpec((1, N, C), lambda b: (b, 0, 0)),
            _row((PPAD, C)),
            pl.BlockSpec((1, N, 1), lambda b: (b, 0, 0)),
            pl.BlockSpec((1, N, 1), lambda b: (b, 0, 0)),
            _row((PPAD, 1)), _row((PPAD, 1)),
            _row((1, C)), _row((1, C)),
            _row((C, C)), _row((C, C)), _row((C, C)), _row((C, C)),
            _row((1, C)),
        ],
        out_specs=[pl.BlockSpec((1, N, C), lambda b: (b, 0, 0)),
                   pl.BlockSpec((1, N, CPAD), lambda b: (b, 0, 0))],
        out_shape=[jax.ShapeDtypeStruct((B, N, C), jnp.float32),
                   jax.ShapeDtypeStruct((B, N, CPAD), jnp.float32)],
        compiler_params=pltpu.CompilerParams(
            dimension_semantics=("arbitrary",)),
    )(x, prm16, m1d, v1d, m1p, v1p, ln1g, ln1b, w_q, w_k, w_v, w_proj, bproj)


def _select_body(h1_ref, m2_ref, v2_ref, ln2g_ref, ln2b_ref, wfc_ref,
                 bfc_ref, wscore_ref, idx_ref):
    b = pl.program_id(0)
    h1 = h1_ref[0]
    ln2 = ((h1 - m2_ref[0]) / jnp.sqrt(v2_ref[0] + 1e-5) * ln2g_ref[...]
           + ln2b_ref[...])
    h2 = h1 + (jnp.dot(ln2, wfc_ref[...],
                       preferred_element_type=jnp.float32) + bfc_ref[...])
    s_col = jnp.dot(h2, wscore_ref[...],
                    preferred_element_type=jnp.float32)

    hi = lax.Precision.HIGHEST
    ii = lax.broadcasted_iota(jnp.int32, (N, N), 0)
    jj = lax.broadcasted_iota(jnp.int32, (N, N), 1)
    ident = (ii == jj).astype(jnp.float32)
    s_row = lax.dot_general(s_col, ident, (((0,), (0,)), ((), ())),
                            preferred_element_type=jnp.float32,
                            precision=hi)
    beats = (s_row > s_col) | ((s_row == s_col) & (jj < ii))
    rank = jnp.dot(beats.astype(jnp.float32), jnp.ones((N, 1), jnp.float32),
                   preferred_element_type=jnp.float32, precision=hi)
    sel = (rank < float(K)).astype(jnp.float32)
    lts = (jj < ii).astype(jnp.float32)
    pos = jnp.dot(lts, sel, preferred_element_type=jnp.float32,
                  precision=hi)
    rr = lax.broadcasted_iota(jnp.int32, (N, K), 1)
    onehot = ((rr == pos.astype(jnp.int32)) & (sel > 0.5)).astype(jnp.float32)
    jcol = lax.broadcasted_iota(jnp.int32, (N, 1), 0).astype(jnp.float32)
    idx_row = lax.dot_general(jcol, onehot, (((0,), (0,)), ((), ())),
                              preferred_element_type=jnp.float32,
                              precision=hi)
    idx_ref[0] = idx_row.astype(jnp.int32) + b * N


def _select_call(h1, m2, v2, ln2g, ln2b, w_fc, bfc, w_score):
    return pl.pallas_call(
        _select_body,
        grid=(B,),
        in_specs=[
            pl.BlockSpec((1, N, C), lambda b: (b, 0, 0)),
            pl.BlockSpec((1, N, 1), lambda b: (b, 0, 0)),
            pl.BlockSpec((1, N, 1), lambda b: (b, 0, 0)),
            _row((1, C)), _row((1, C)), _row((C, C)), _row((1, C)),
            _row((C, 1)),
        ],
        out_specs=pl.BlockSpec((1, 1, K), lambda b: (b, 0, 0)),
        out_shape=jax.ShapeDtypeStruct((B, 1, K), jnp.int32),
        compiler_params=pltpu.CompilerParams(
            dimension_semantics=("arbitrary",)),
    )(h1, m2, v2, ln2g, ln2b, w_fc, bfc, w_score)


_NC, _NS = 2, 16
_NW = _NC * _NS
ROWS_W = (B * K) // _NW
CHUNK = 128
NCHUNK = ROWS_W // CHUNK


@functools.cache
def _make_sc_gather():
    @functools.partial(
        pl.kernel,
        out_type=jax.ShapeDtypeStruct((B * K, CPAD), jnp.float32),
        mesh=plsc.VectorSubcoreMesh(core_axis_name="c", subcore_axis_name="s",
                                    num_cores=_NC, num_subcores=_NS),
        scratch_types=[
            pltpu.VMEM((NCHUNK, CHUNK), jnp.int32),
            pltpu.VMEM((ROWS_W, CPAD), jnp.float32),
            pltpu.SemaphoreType.DMA,
        ],
    )
    def _sc_gather(x_hbm, idx_hbm, out_hbm, idx_v, rows_v, sem):
        wid = lax.axis_index("s") * _NC + lax.axis_index("c")
        base = wid * ROWS_W
        pltpu.sync_copy(idx_hbm.at[wid], idx_v)
        copies = [
            pltpu.async_copy(x_hbm.at[idx_v.at[j]],
                             rows_v.at[pl.ds(j * CHUNK, CHUNK)], sem)
            for j in range(NCHUNK)
        ]
        for c in copies:
            c.wait()
        pltpu.sync_copy(rows_v, out_hbm.at[pl.ds(base, ROWS_W)])

    return _sc_gather


def kernel(x, prompt, ln1_g, ln1_b, w_qkv, w_proj, b_proj, ln2_g, ln2_b,
           w_fc, b_fc, w_score, b_score):
    xp_ref_order = jnp.concatenate(
        [jnp.broadcast_to(prompt, (B, NUM_PROMPT, C)), x], axis=1)
    m1 = jnp.mean(xp_ref_order, axis=-1, keepdims=True)
    v1 = jnp.var(xp_ref_order, axis=-1, keepdims=True)
    m1d, v1d = m1[:, NUM_PROMPT:], v1[:, NUM_PROMPT:]
    pad = jnp.zeros((PPAD - NUM_PROMPT, 1), jnp.float32)
    m1p = jnp.concatenate([m1[0, :NUM_PROMPT], pad], axis=0)
    v1p = jnp.concatenate([v1[0, :NUM_PROMPT], pad + 1.0], axis=0)
    prm16 = jnp.concatenate(
        [prompt[0], jnp.zeros((PPAD - NUM_PROMPT, C), jnp.float32)], axis=0)

    h1, x_pad = _attn_call(x, prm16, m1d, v1d, m1p, v1p, ln1_g[None],
                           ln1_b[None], w_qkv[:, :C], w_qkv[:, C:2 * C],
                           w_qkv[:, 2 * C:], w_proj, b_proj[None])

    m2 = jnp.mean(h1, axis=-1, keepdims=True)
    v2 = jnp.var(h1, axis=-1, keepdims=True)
    idx = _select_call(h1, m2, v2, ln2_g[None], ln2_b[None], w_fc,
                       b_fc[None], w_score)

    out = _make_sc_gather()(x_pad.reshape(B * N, CPAD),
                            idx.reshape(B, NCHUNK, CHUNK))
    return out[:, :C].reshape(B, K, C)
